# baseline (speedup 1.0000x reference)
import functools

import jax
import jax.numpy as jnp
from jax import lax
from jax.experimental import pallas as pl
from jax.experimental.pallas import tpu as pltpu
from jax.experimental.pallas import tpu_sc as plsc

NC = 2
NS = 16
L = 16
NW = NC * NS
R, C = 2048, 2048
N = R * C
ROWS_W = R // NW
PH1 = R // NS
KEEP = 32
SCH = 8
P1N = (PH1 - KEEP) // SCH
P2N = (ROWS_W - KEEP) // SCH
RG = KEEP // 8
U = 8

_ABS_MASK = 0x7FFFFFFF
_SIGN_MASK = -0x80000000

_MESH = plsc.VectorSubcoreMesh(
    core_axis_name="c", subcore_axis_name="s", num_cores=NC, num_subcores=NS
)
_PARAMS = pltpu.CompilerParams(
    needs_layout_passes=False, use_tc_tiling_on_sc=True
)


def _lane_sum(v, tmp):
    lanes = lax.iota(jnp.int32, L)
    for s in (1, 2, 4, 8):
        tmp[...] = v
        v = v + plsc.load_gather(tmp, [lanes ^ s])
    return v


def _body(w_hbm, table_hbm, alpha_hbm, out_hbm,
          res, sb0, sb1, shared, pstat, table_v, alpha_v, svq, tmp_v,
          sem_rin, sem_rout, si0, si1, so0, so1):
    c_ax = lax.axis_index("c")
    s_ax = lax.axis_index("s")
    r0 = s_ax * PH1 + c_ax * ROWS_W
    n0 = s_ax * PH1 + (1 - c_ax) * ROWS_W

    pltpu.make_async_copy(w_hbm.at[pl.ds(r0, KEEP), :], res, sem_rin).start()

    def p1row(ci):
        return pl.multiple_of(
            s_ax * PH1 + ((c_ax * ROWS_W + KEEP + SCH * ci) & (PH1 - 1)), SCH
        )

    def _p1copy(ci, buf, sem):
        return pltpu.make_async_copy(
            w_hbm.at[pl.ds(p1row(ci), SCH), :], buf, sem
        )

    _p1copy(0, sb0, si0).start()
    _p1copy(1, sb1, si1).start()

    pltpu.sync_copy(table_hbm, table_v)
    pltpu.sync_copy(alpha_hbm, alpha_v)

    def _acc4(buf, carry):
        def sweep(o, carry2):
            s2, q2 = carry2
            for r in range(SCH):
                xs = [buf[r, pl.ds(o + u * L, L)] for u in range(U)]
                for u in range(0, U, 2):
                    s2 = s2 + (xs[u] + xs[u + 1])
                    q2 = q2 + (xs[u] * xs[u] + xs[u + 1] * xs[u + 1])
            return s2, q2

        return plsc.parallel_loop(0, C, step=U * L, carry=carry)(sweep)

    def p1loop(j2, carry):
        c0 = 2 * j2
        _p1copy(c0, sb0, si0).wait()
        carry = _acc4(sb0, carry)

        @pl.when(c0 + 2 < P1N)
        def _():
            _p1copy(c0 + 2, sb0, si0).start()

        _p1copy(c0 + 1, sb1, si1).wait()
        carry = _acc4(sb1, carry)

        @pl.when(c0 + 3 < P1N)
        def _():
            _p1copy(c0 + 3, sb1, si1).start()

        return carry

    zero = jnp.zeros((L,), jnp.float32)
    s, q = lax.fori_loop(0, P1N // 2, p1loop, (zero, zero))

    def _p2in(sc, buf, sem):
        return pltpu.make_async_copy(
            w_hbm.at[pl.ds(r0 + KEEP + SCH * sc, SCH), :], buf, sem
        )

    _p2in(0, sb0, si0).start()
    _p2in(1, sb1, si1).start()

    pltpu.make_async_copy(w_hbm.at[pl.ds(r0, KEEP), :], res, sem_rin).wait()

    def accres(g, carry):
        def sweep(o, carry2):
            s2, q2 = carry2
            for r in range(8):
                xs = [res[g * 8 + r, pl.ds(o + u * L, L)] for u in range(U)]
                for u in range(0, U, 2):
                    s2 = s2 + (xs[u] + xs[u + 1])
                    q2 = q2 + (xs[u] * xs[u] + xs[u + 1] * xs[u + 1])
            return s2, q2

        return plsc.parallel_loop(0, C, step=U * L, carry=carry)(sweep)

    s, q = lax.fori_loop(0, RG, accres, (s, q))

    svq[pl.ds(0, L)] = s
    svq[pl.ds(L, L)] = q
    pltpu.sync_copy(svq, shared.at[pl.ds(s_ax * 2 * L, 2 * L)])
    plsc.subcore_barrier()
    pltpu.sync_copy(shared, pstat)

    def comb(i, carry):
        s2, q2 = carry
        return (s2 + pstat[pl.ds(i * 2 * L, L)],
                q2 + pstat[pl.ds(i * 2 * L + L, L)])

    s, q = lax.fori_loop(0, NS, comb, (zero, zero))
    tot = _lane_sum(s, tmp_v)
    totq = _lane_sum(q, tmp_v)
    mean = tot * jnp.float32(1.0 / N)
    var = (totq - jnp.float32(N) * mean * mean) * jnp.float32(1.0 / (N - 1))
    vb = plsc.bitcast(var, jnp.int32)
    magic = jnp.full((L,), 0x5F3759DF, dtype=jnp.int32)
    y = plsc.bitcast(magic - lax.shift_right_logical(vb, 1), jnp.float32)
    for _ in range(3):
        y = y * (jnp.float32(1.5) - jnp.float32(0.5) * var * y * y)
    s7 = (y / alpha_v[...]) * jnp.float32(7.0)
    m7 = mean * s7

    abs_mask = jnp.full((L,), _ABS_MASK, dtype=jnp.int32)
    sign_mask = jnp.full((L,), _SIGN_MASK, dtype=jnp.int32)
    seven_i = jnp.full((L,), 7, dtype=jnp.int32)
    seven_f = jnp.full((L,), 7.0, dtype=jnp.float32)
    half8 = jnp.full((L,), 7.5, dtype=jnp.float32)

    def _quant(x):
        z = x * s7 - m7
        zi = plsc.bitcast(z, jnp.int32)
        az = plsc.bitcast(zi & abs_mask, jnp.float32)
        az = jnp.minimum(az, seven_f)
        idx = seven_i - (half8 - az).astype(jnp.int32)
        g = plsc.load_gather(table_v, [idx])
        return plsc.bitcast(
            plsc.bitcast(g, jnp.int32) ^ (zi & sign_mask), jnp.float32
        )

    def _xf4(buf):
        def sweep(o):
            for r in range(SCH):
                buf[r, pl.ds(o, L)] = _quant(buf[r, pl.ds(o, L)])

        plsc.parallel_loop(0, C, step=L, unroll=U)(sweep)

    def _p2out(sc, buf, sem):
        return pltpu.make_async_copy(
            buf, out_hbm.at[pl.ds(r0 + KEEP + SCH * sc, SCH), :], sem
        )

    _p2in(0, sb0, si0).wait()
    _xf4(sb0)
    _p2out(0, sb0, so0).start()
    _p2in(1, sb1, si1).wait()
    _xf4(sb1)
    _p2out(1, sb1, so1).start()

    def resg(g, carry):
        def sweep(o):
            for r in range(8):
                res[g * 8 + r, pl.ds(o, L)] = _quant(res[g * 8 + r, pl.ds(o, L)])

        plsc.parallel_loop(0, C, step=L, unroll=U)(sweep)
        pltpu.make_async_copy(
            res.at[pl.ds(g * 8, 8), :],
            out_hbm.at[pl.ds(r0 + g * 8, 8), :],
            sem_rout,
        ).start()
        return carry

    lax.fori_loop(0, RG, resg, 0)

    _p2out(0, sb0, so0).wait()
    _p2in(2, sb0, si0).start()
    _p2out(1, sb1, so1).wait()
    _p2in(3, sb1, si1).start()

    _p2in(2, sb0, si0).wait()
    _xf4(sb0)
    _p2out(2, sb0, so0).start()
    _p2in(3, sb1, si1).wait()
    _xf4(sb1)
    _p2out(3, sb1, so1).start()

    _p2out(2, sb0, so0).wait()
    _p2out(3, sb1, so1).wait()
    for g in range(RG):
        pltpu.make_async_copy(
            res.at[pl.ds(g * 8, 8), :],
            out_hbm.at[pl.ds(r0 + g * 8, 8), :],
            sem_rout,
        ).wait()


@functools.partial(
    pl.kernel,
    out_type=jax.ShapeDtypeStruct((R, C), jnp.float32),
    mesh=_MESH,
    compiler_params=_PARAMS,
    scratch_types=[
        pltpu.VMEM((KEEP, C), jnp.float32),
        pltpu.VMEM((SCH, C), jnp.float32),
        pltpu.VMEM((SCH, C), jnp.float32),
        pltpu.VMEM_SHARED((NS * 2 * L,), jnp.float32),
        pltpu.VMEM((NS * 2 * L,), jnp.float32),
        pltpu.VMEM((L,), jnp.float32),
        pltpu.VMEM((L,), jnp.float32),
        pltpu.VMEM((2 * L,), jnp.float32),
        pltpu.VMEM((L,), jnp.float32),
        pltpu.SemaphoreType.DMA,
        pltpu.SemaphoreType.DMA,
        pltpu.SemaphoreType.DMA,
        pltpu.SemaphoreType.DMA,
        pltpu.SemaphoreType.DMA,
        pltpu.SemaphoreType.DMA,
    ],
)
def _merged_call(w_hbm, table_hbm, alpha_hbm, out_hbm,
                 res, sb0, sb1, shared, pstat, table_v, alpha_v, svq, tmp_v,
                 sem_rin, sem_rout, si0, si1, so0, so1):
    _body(w_hbm, table_hbm, alpha_hbm, out_hbm,
          res, sb0, sb1, shared, pstat, table_v, alpha_v, svq, tmp_v,
          sem_rin, sem_rout, si0, si1, so0, so1)


def kernel(weight, wgt_alpha):
    grid = jnp.linspace(0.0, 1.0, 8, dtype=jnp.float32) * 1.0
    table = grid * wgt_alpha.astype(jnp.float32)
    table16 = jnp.concatenate([table, jnp.zeros((8,), jnp.float32)])
    alpha16 = jnp.full((L,), wgt_alpha, dtype=jnp.float32)
    return _merged_call(weight, table16, alpha16)


# speedup vs baseline: 1.4106x; 1.4106x over previous
import functools

import jax
import jax.numpy as jnp
from jax import lax
from jax.experimental import pallas as pl
from jax.experimental.pallas import tpu as pltpu
from jax.experimental.pallas import tpu_sc as plsc

NC = 2
NS = 16
L = 16
NW = NC * NS
R, C = 2048, 2048
N = R * C
ROWS_W = R // NW
CROWS = 8
NCHUNK = ROWS_W // CROWS
XROWS = 4
XNCH = ROWS_W // XROWS
XD = 4
U = 8

_ABS_MASK = 0x7FFFFFFF
_SIGN_MASK = -0x80000000

_MESH = plsc.VectorSubcoreMesh(
    core_axis_name="c", subcore_axis_name="s", num_cores=NC, num_subcores=NS
)

_PARAMS = pltpu.CompilerParams(
    needs_layout_passes=False, use_tc_tiling_on_sc=True
)


def _wid():
    return lax.axis_index("s") * NC + lax.axis_index("c")


TCROWS = 256


def _tc_stats_body(w_ref, stat_ref, acc_ref):
    i = pl.program_id(0)
    x = w_ref[...]
    s = jnp.sum(x)
    q = jnp.sum(x * x)

    @pl.when(i == 0)
    def _():
        acc_ref[0] = jnp.float32(0.0)
        acc_ref[1] = jnp.float32(0.0)

    acc_ref[0] = acc_ref[0] + s
    acc_ref[1] = acc_ref[1] + q

    @pl.when(i == pl.num_programs(0) - 1)
    def _():
        for j in range(L):
            stat_ref[j] = acc_ref[0]
            stat_ref[L + j] = acc_ref[1]


_tc_stats = pl.pallas_call(
    _tc_stats_body,
    grid=(R // TCROWS,),
    in_specs=[pl.BlockSpec((TCROWS, C), lambda i: (i, 0))],
    out_specs=pl.BlockSpec(memory_space=pltpu.SMEM),
    out_shape=jax.ShapeDtypeStruct((2 * L,), jnp.float32),
    scratch_shapes=[pltpu.SMEM((2,), jnp.float32)],
    compiler_params=pltpu.CompilerParams(dimension_semantics=("arbitrary",)),
)


def _xform_body(w_hbm, stats_hbm, table_hbm, alpha_hbm, out_hbm,
                stat_v, table_v, alpha_v,
                in0, in1, in2, in3, out0, out1, out2, out3,
                si0, si1, si2, si3, so0, so1, so2, so3):
    row0 = _wid() * ROWS_W

    def _xin_copy(c, buf, sem):
        return pltpu.make_async_copy(
            w_hbm.at[pl.ds(row0 + c * XROWS, XROWS), :], buf, sem
        )

    ins = [in0, in1, in2, in3]
    outs = [out0, out1, out2, out3]
    isems = [si0, si1, si2, si3]
    osems = [so0, so1, so2, so3]
    for b in range(XD):
        _xin_copy(b, ins[b], isems[b]).start()

    pltpu.sync_copy(stats_hbm, stat_v)
    pltpu.sync_copy(table_hbm, table_v)
    pltpu.sync_copy(alpha_hbm, alpha_v)

    tot = stat_v[pl.ds(0, L)]
    totq = stat_v[pl.ds(L, L)]
    mean = tot * jnp.float32(1.0 / N)
    var = (totq - jnp.float32(N) * mean * mean) * jnp.float32(1.0 / (N - 1))
    vb = plsc.bitcast(var, jnp.int32)
    magic = jnp.full((L,), 0x5F3759DF, dtype=jnp.int32)
    y = plsc.bitcast(magic - lax.shift_right_logical(vb, 1), jnp.float32)
    for _ in range(3):
        y = y * (jnp.float32(1.5) - jnp.float32(0.5) * var * y * y)
    s7 = (y / alpha_v[...]) * jnp.float32(7.0)
    m7 = mean * s7

    abs_mask = jnp.full((L,), _ABS_MASK, dtype=jnp.int32)
    sign_mask = jnp.full((L,), _SIGN_MASK, dtype=jnp.int32)
    seven_i = jnp.full((L,), 7, dtype=jnp.int32)
    seven_f = jnp.full((L,), 7.0, dtype=jnp.float32)
    half8 = jnp.full((L,), 7.5, dtype=jnp.float32)

    def _out_copy(c, buf, sem):
        return pltpu.make_async_copy(
            buf, out_hbm.at[pl.ds(row0 + c * XROWS, XROWS), :], sem
        )

    def _xf(ibuf, obuf):
        def vec_body(o):
            for r in range(XROWS):
                x = ibuf[r, pl.ds(o, L)]
                z = x * s7 - m7
                zi = plsc.bitcast(z, jnp.int32)
                az = plsc.bitcast(zi & abs_mask, jnp.float32)
                az = jnp.minimum(az, seven_f)
                idx = seven_i - (half8 - az).astype(jnp.int32)
                g = plsc.load_gather(table_v, [idx])
                gi = plsc.bitcast(g, jnp.int32) ^ (zi & sign_mask)
                obuf[r, pl.ds(o, L)] = plsc.bitcast(gi, jnp.float32)

        plsc.parallel_loop(0, C, step=L, unroll=U)(vec_body)

    def ring(j4, carry):
        for b in range(XD):
            c = XD * j4 + b
            _xin_copy(c, ins[b], isems[b]).wait()

            @pl.when(j4 > 0)
            def _():
                _out_copy(c - XD, outs[b], osems[b]).wait()

            _xf(ins[b], outs[b])
            _out_copy(c, outs[b], osems[b]).start()

            @pl.when(c + XD < XNCH)
            def _():
                _xin_copy(c + XD, ins[b], isems[b]).start()

        return carry

    lax.fori_loop(0, XNCH // XD, ring, 0)
    for b in range(XD):
        _out_copy(XNCH - XD + b, outs[b], osems[b]).wait()


@functools.partial(
    pl.kernel,
    out_type=jax.ShapeDtypeStruct((R, C), jnp.float32),
    mesh=_MESH,
    compiler_params=_PARAMS,
    scratch_types=[
        pltpu.VMEM((2 * L,), jnp.float32),
        pltpu.VMEM((L,), jnp.float32),
        pltpu.VMEM((L,), jnp.float32),
    ]
    + [pltpu.VMEM((XROWS, C), jnp.float32)] * (2 * XD)
    + [pltpu.SemaphoreType.DMA] * (2 * XD),
)
def _xform_call(w_hbm, stats_hbm, table_hbm, alpha_hbm, out_hbm,
                stat_v, table_v, alpha_v,
                in0, in1, in2, in3, out0, out1, out2, out3,
                si0, si1, si2, si3, so0, so1, so2, so3):
    _xform_body(w_hbm, stats_hbm, table_hbm, alpha_hbm, out_hbm,
                stat_v, table_v, alpha_v,
                in0, in1, in2, in3, out0, out1, out2, out3,
                si0, si1, si2, si3, so0, so1, so2, so3)


def kernel(weight, wgt_alpha):
    grid = jnp.linspace(0.0, 1.0, 8, dtype=jnp.float32) * 1.0
    table = grid * wgt_alpha.astype(jnp.float32)
    table16 = jnp.concatenate([table, jnp.zeros((8,), jnp.float32)])
    alpha16 = jnp.full((L,), wgt_alpha, dtype=jnp.float32)
    stats = _tc_stats(weight)
    out = _xform_call(weight, stats, table16, alpha16)
    return out


# speedup vs baseline: 1.4633x; 1.0373x over previous
import functools

import jax
import jax.numpy as jnp
from jax import lax
from jax.experimental import pallas as pl
from jax.experimental.pallas import tpu as pltpu
from jax.experimental.pallas import tpu_sc as plsc

NC = 2
NS = 16
L = 16
NW = NC * NS
R, C = 2048, 2048
N = R * C
ROWS_W = R // NW
CROWS = 8
NCHUNK = ROWS_W // CROWS
XROWS = 4
XNCH = ROWS_W // XROWS
XD = 4
U = 8

_ABS_MASK = 0x7FFFFFFF
_SIGN_MASK = -0x80000000

_MESH = plsc.VectorSubcoreMesh(
    core_axis_name="c", subcore_axis_name="s", num_cores=NC, num_subcores=NS
)

_PARAMS = pltpu.CompilerParams(
    needs_layout_passes=False, use_tc_tiling_on_sc=True
)


def _wid():
    return lax.axis_index("s") * NC + lax.axis_index("c")


TCROWS = 512


def _tc_stats_body(w_ref, stat_ref, acc_ref):
    i = pl.program_id(0)
    x = w_ref[...]
    s = jnp.sum(x)
    q = jnp.sum(x * x)

    @pl.when(i == 0)
    def _():
        acc_ref[0] = jnp.float32(0.0)
        acc_ref[1] = jnp.float32(0.0)

    acc_ref[0] = acc_ref[0] + s
    acc_ref[1] = acc_ref[1] + q

    @pl.when(i == pl.num_programs(0) - 1)
    def _():
        for j in range(L):
            stat_ref[j] = acc_ref[0]
            stat_ref[L + j] = acc_ref[1]


_tc_stats = pl.pallas_call(
    _tc_stats_body,
    grid=(R // TCROWS,),
    in_specs=[pl.BlockSpec((TCROWS, C), lambda i: (i, 0))],
    out_specs=pl.BlockSpec(memory_space=pltpu.SMEM),
    out_shape=jax.ShapeDtypeStruct((2 * L,), jnp.float32),
    scratch_shapes=[pltpu.SMEM((2,), jnp.float32)],
    compiler_params=pltpu.CompilerParams(dimension_semantics=("arbitrary",)),
)


def _xform_body(w_hbm, stats_hbm, table_hbm, alpha_hbm, out_hbm,
                stat_v, table_v, alpha_v,
                in0, in1, in2, in3, out0, out1, out2, out3,
                si0, si1, si2, si3, so0, so1, so2, so3):
    row0 = _wid() * ROWS_W

    def _xin_copy(c, buf, sem):
        return pltpu.make_async_copy(
            w_hbm.at[pl.ds(row0 + c * XROWS, XROWS), :], buf, sem
        )

    ins = [in0, in1, in2, in3]
    outs = [out0, out1, out2, out3]
    isems = [si0, si1, si2, si3]
    osems = [so0, so1, so2, so3]
    for b in range(XD):
        _xin_copy(b, ins[b], isems[b]).start()

    pltpu.sync_copy(stats_hbm, stat_v)
    pltpu.sync_copy(table_hbm, table_v)
    pltpu.sync_copy(alpha_hbm, alpha_v)

    tot = stat_v[pl.ds(0, L)]
    totq = stat_v[pl.ds(L, L)]
    mean = tot * jnp.float32(1.0 / N)
    var = (totq - jnp.float32(N) * mean * mean) * jnp.float32(1.0 / (N - 1))
    vb = plsc.bitcast(var, jnp.int32)
    magic = jnp.full((L,), 0x5F3759DF, dtype=jnp.int32)
    y = plsc.bitcast(magic - lax.shift_right_logical(vb, 1), jnp.float32)
    for _ in range(3):
        y = y * (jnp.float32(1.5) - jnp.float32(0.5) * var * y * y)
    s7 = (y / alpha_v[...]) * jnp.float32(7.0)
    m7 = mean * s7

    abs_mask = jnp.full((L,), _ABS_MASK, dtype=jnp.int32)
    sign_mask = jnp.full((L,), _SIGN_MASK, dtype=jnp.int32)
    seven_i = jnp.full((L,), 7, dtype=jnp.int32)
    seven_f = jnp.full((L,), 7.0, dtype=jnp.float32)
    half8 = jnp.full((L,), 7.5, dtype=jnp.float32)

    def _out_copy(c, buf, sem):
        return pltpu.make_async_copy(
            buf, out_hbm.at[pl.ds(row0 + c * XROWS, XROWS), :], sem
        )

    def _xf(ibuf, obuf):
        def vec_body(o):
            for r in range(XROWS):
                x = ibuf[r, pl.ds(o, L)]
                z = x * s7 - m7
                zi = plsc.bitcast(z, jnp.int32)
                az = plsc.bitcast(zi & abs_mask, jnp.float32)
                az = jnp.minimum(az, seven_f)
                idx = seven_i - (half8 - az).astype(jnp.int32)
                g = plsc.load_gather(table_v, [idx])
                gi = plsc.bitcast(g, jnp.int32) ^ (zi & sign_mask)
                obuf[r, pl.ds(o, L)] = plsc.bitcast(gi, jnp.float32)

        plsc.parallel_loop(0, C, step=L, unroll=U)(vec_body)

    def ring(j4, carry):
        for b in range(XD):
            c = XD * j4 + b
            _xin_copy(c, ins[b], isems[b]).wait()

            @pl.when(j4 > 0)
            def _():
                _out_copy(c - XD, outs[b], osems[b]).wait()

            _xf(ins[b], outs[b])
            _out_copy(c, outs[b], osems[b]).start()

            @pl.when(c + XD < XNCH)
            def _():
                _xin_copy(c + XD, ins[b], isems[b]).start()

        return carry

    lax.fori_loop(0, XNCH // XD, ring, 0)
    for b in range(XD):
        _out_copy(XNCH - XD + b, outs[b], osems[b]).wait()


@functools.partial(
    pl.kernel,
    out_type=jax.ShapeDtypeStruct((R, C), jnp.float32),
    mesh=_MESH,
    compiler_params=_PARAMS,
    scratch_types=[
        pltpu.VMEM((2 * L,), jnp.float32),
        pltpu.VMEM((L,), jnp.float32),
        pltpu.VMEM((L,), jnp.float32),
    ]
    + [pltpu.VMEM((XROWS, C), jnp.float32)] * (2 * XD)
    + [pltpu.SemaphoreType.DMA] * (2 * XD),
)
def _xform_call(w_hbm, stats_hbm, table_hbm, alpha_hbm, out_hbm,
                stat_v, table_v, alpha_v,
                in0, in1, in2, in3, out0, out1, out2, out3,
                si0, si1, si2, si3, so0, so1, so2, so3):
    _xform_body(w_hbm, stats_hbm, table_hbm, alpha_hbm, out_hbm,
                stat_v, table_v, alpha_v,
                in0, in1, in2, in3, out0, out1, out2, out3,
                si0, si1, si2, si3, so0, so1, so2, so3)


def kernel(weight, wgt_alpha):
    grid = jnp.linspace(0.0, 1.0, 8, dtype=jnp.float32) * 1.0
    table = grid * wgt_alpha.astype(jnp.float32)
    table16 = jnp.concatenate([table, jnp.zeros((8,), jnp.float32)])
    alpha16 = jnp.full((L,), wgt_alpha, dtype=jnp.float32)
    stats = _tc_stats(weight)
    out = _xform_call(weight, stats, table16, alpha16)
    return out
